# baseline (speedup 1.0000x reference)
import jax
import jax.numpy as jnp
from jax import lax
from jax.experimental import pallas as pl
from jax.experimental.pallas import tpu as pltpu
from jax.experimental.pallas import tpu_sc as plsc

N = 10000
E = 320000
D = 128
OUT = 47
NPAD = 10240
NC, NS = 2, 16
NW = NC * NS
CHUNK = 128
NBUF = 2
CH = 80
EPAD = NW * CH * CHUNK
DUMMY = N
RPT = NPAD // NS


def _make_sc_agg(width):
  mesh = plsc.VectorSubcoreMesh(core_axis_name="c", subcore_axis_name="s")
  out_type = [jax.ShapeDtypeStruct((NC, NPAD, width), jnp.float32)]
  scratch = (
      [pltpu.VMEM((2, CHUNK), jnp.int32)] * NBUF +
      [pltpu.VMEM((CHUNK, width), jnp.float32)] * NBUF +
      [pltpu.VMEM_SHARED((NPAD, width), jnp.float32)] +
      [pltpu.SemaphoreType.DMA] * NBUF
  )

  def body(hn, idxw, zrs, parts, *scr):
    idx_v = scr[0:NBUF]
    rows_v = scr[NBUF:2 * NBUF]
    agg_sh = scr[2 * NBUF]
    sem = scr[2 * NBUF + 1:2 * NBUF + 1 + NBUF]
    cid = lax.axis_index("c")
    sid = lax.axis_index("s")
    wid = cid * NS + sid

    r0 = sid * RPT
    pltpu.sync_copy(zrs, agg_sh.at[pl.ds(r0, RPT)])

    plsc.subcore_barrier()

    @pl.loop(0, CH // NBUF)
    def _(j):
      cps = []
      for b in range(NBUF):
        c = j * NBUF + b
        pltpu.sync_copy(idxw.at[wid, c], idx_v[b])
        cps.append(pltpu.async_copy(hn.at[idx_v[b].at[0]], rows_v[b], sem[b]))
      for b in range(NBUF):
        cps[b].wait()
        pltpu.sync_copy(rows_v[b], agg_sh.at[idx_v[b].at[1]], add=True)

    plsc.subcore_barrier()

    pltpu.sync_copy(agg_sh.at[pl.ds(r0, RPT)], parts.at[cid, pl.ds(r0, RPT)])

  params = pltpu.CompilerParams(use_tc_tiling_on_sc=False)
  return pl.kernel(body, out_type=out_type, mesh=mesh, scratch_types=scratch,
                   compiler_params=params)


DE = 144
_sc_agg_ext = _make_sc_agg(DE)
_sc_agg = _make_sc_agg(D)


RB = 1280
GRID = NPAD // RB


def _pre_body(x_ref, w_ref, o_ref):
  mm = jnp.dot(x_ref[...], w_ref[...], preferred_element_type=jnp.float32)
  o_ref[:, :D] = mm
  col = lax.broadcasted_iota(jnp.int32, (RB, DE - D), 1)
  o_ref[:, D:] = jnp.where(col == 0, 1.0, 0.0).astype(jnp.float32)


_tc_pre = pl.pallas_call(
    _pre_body,
    grid=(GRID,),
    in_specs=[pl.BlockSpec((RB, D), lambda i: (i, 0)),
              pl.BlockSpec((D, D), lambda i: (0, 0))],
    out_specs=pl.BlockSpec((RB, DE), lambda i: (i, 0)),
    out_shape=jax.ShapeDtypeStruct((NPAD, DE), jnp.float32),
)


_row = pl.BlockSpec((RB, D), lambda i: (i, 0))
_prow = pl.BlockSpec((NC, RB, D), lambda i: (0, i, 0))
_perow = pl.BlockSpec((NC, RB, DE), lambda i: (0, i, 0))
_irow = pl.BlockSpec((RB, 16), lambda i: (i, 0))
_wfull = pl.BlockSpec((D, D), lambda i: (0, 0))
_bfull = pl.BlockSpec((1, D), lambda i: (0, 0))


def _tcb1_body(x_ref, p_ref, ws, b, wn, h_out, hn_out, inv_out):
  p = p_ref[...]
  deg = jnp.maximum(p[0, :, D:D + 16] + p[1, :, D:D + 16], 1.0)
  inv = 1.0 / deg
  inv_out[...] = inv
  mean = (p[0, :, :D] + p[1, :, :D]) * inv[:, 0:1]
  h = jnp.dot(x_ref[...], ws[...], preferred_element_type=jnp.float32)
  h = jnp.maximum(h + mean + b[...], 0.0)
  h_out[...] = h
  hn_out[...] = jnp.dot(h, wn[...], preferred_element_type=jnp.float32)


_tcb1 = pl.pallas_call(
    _tcb1_body,
    grid=(GRID,),
    in_specs=[_row, _perow, _wfull, _bfull, _wfull],
    out_specs=[_row, _row, _irow],
    out_shape=[jax.ShapeDtypeStruct((NPAD, D), jnp.float32),
               jax.ShapeDtypeStruct((NPAD, D), jnp.float32),
               jax.ShapeDtypeStruct((NPAD, 16), jnp.float32)],
)


def _tcb2_body(x_ref, p_ref, inv_ref, ws, b, wn, h_out, hn_out):
  inv = inv_ref[...][:, 0:1]
  p = p_ref[...]
  mean = (p[0] + p[1]) * inv
  h = jnp.dot(x_ref[...], ws[...], preferred_element_type=jnp.float32)
  h = jnp.maximum(h + mean + b[...], 0.0)
  h_out[...] = h
  hn_out[...] = jnp.dot(h, wn[...], preferred_element_type=jnp.float32)


_tcb2 = pl.pallas_call(
    _tcb2_body,
    grid=(GRID,),
    in_specs=[_row, _prow, _irow, _wfull, _bfull, _wfull],
    out_specs=[_row, _row],
    out_shape=[jax.ShapeDtypeStruct((NPAD, D), jnp.float32),
               jax.ShapeDtypeStruct((NPAD, D), jnp.float32)],
)


def _tcc_body(x_ref, p_ref, inv_ref, ws, b, o_ref):
  inv = inv_ref[...][:, 0:1]
  p = p_ref[...]
  mean = (p[0] + p[1]) * inv
  h = jnp.dot(x_ref[...], ws[...], preferred_element_type=jnp.float32)
  o_ref[...] = h + mean + b[...]


_tcc = pl.pallas_call(
    _tcc_body,
    grid=(GRID,),
    in_specs=[_row, _prow, _irow, _wfull, _bfull],
    out_specs=_row,
    out_shape=jax.ShapeDtypeStruct((NPAD, D), jnp.float32),
)


def kernel(x, edge_index, W_self1, W_neigh1, b1, W_self2, W_neigh2, b2,
           W_self3, W_neigh3, b3):
  src = edge_index[0].astype(jnp.int32)
  dst = edge_index[1].astype(jnp.int32)
  pad = EPAD - E
  pad_dst = DUMMY + jnp.arange(pad, dtype=jnp.int32) % (NPAD - N)
  srcw = jnp.concatenate([src, jnp.zeros((pad,), jnp.int32)]).reshape(NW, CH, 1, CHUNK)
  dstw = jnp.concatenate([dst, pad_dst]).reshape(NW, CH, 1, CHUNK)
  idxw = jnp.concatenate([srcw, dstw], axis=2)
  z_ext = jnp.zeros((RPT, DE), jnp.float32)
  z_d = jnp.zeros((RPT, D), jnp.float32)
  xp = jnp.zeros((NPAD, D), jnp.float32).at[:N].set(x)
  wn3p = jnp.zeros((D, D), jnp.float32).at[:, :OUT].set(W_neigh3)
  ws3p = jnp.zeros((D, D), jnp.float32).at[:, :OUT].set(W_self3)
  b1r = b1.reshape(1, D)
  b2r = b2.reshape(1, D)
  b3p = jnp.zeros((1, D), jnp.float32).at[0, :OUT].set(b3)
  hn1 = _tc_pre(xp, W_neigh1)
  parts1, = _sc_agg_ext(hn1, idxw, z_ext)
  h1, hn2, inv = _tcb1(xp, parts1, W_self1, b1r, W_neigh2)
  parts2, = _sc_agg(hn2, idxw, z_d)
  h2, hn3 = _tcb2(h1, parts2, inv, W_self2, b2r, wn3p)
  parts3, = _sc_agg(hn3, idxw, z_d)
  outp = _tcc(h2, parts3, inv, ws3p, b3p)
  return outp[:N, :OUT]


# speedup vs baseline: 1.4354x; 1.4354x over previous
import jax
import jax.numpy as jnp
from jax import lax
from jax.experimental import pallas as pl
from jax.experimental.pallas import tpu as pltpu
from jax.experimental.pallas import tpu_sc as plsc

N = 10000
E = 320000
D = 128
OUT = 47
NPAD = 10240
NC, NS = 2, 16
NW = NC * NS
CHUNK = 128
CH = -(-E // (NW * CHUNK))
EPAD = NW * CH * CHUNK
DUMMY = N
RPT = NPAD // NS
ZR = 32


def _make_sc_agg(width):
  mesh = plsc.VectorSubcoreMesh(core_axis_name="c", subcore_axis_name="s")
  out_type = [jax.ShapeDtypeStruct((NC, NPAD, width), jnp.float32)]
  scratch = [
      pltpu.VMEM((CHUNK,), jnp.int32),
      pltpu.VMEM((CHUNK,), jnp.int32),
      pltpu.VMEM((CHUNK, width), jnp.float32),
      pltpu.VMEM((ZR, width), jnp.float32),
      pltpu.VMEM_SHARED((NPAD, width), jnp.float32),
      pltpu.SemaphoreType.DMA,
  ]

  def body(hn, srcw, dstw, parts, src_v, dst_v, rows_v, zbuf, agg_sh, sem):
    cid = lax.axis_index("c")
    sid = lax.axis_index("s")
    wid = cid * NS + sid
    zero16 = jnp.zeros((16,), jnp.float32)

    @pl.loop(0, ZR)
    def _(i):
      for j in range(width // 16):
        zbuf[i, pl.ds(j * 16, 16)] = zero16

    r0 = sid * RPT

    @pl.loop(0, RPT // ZR)
    def _(k):
      pltpu.sync_copy(zbuf, agg_sh.at[pl.ds(r0 + k * ZR, ZR)])

    plsc.subcore_barrier()

    @pl.loop(0, CH)
    def _(j):
      pltpu.sync_copy(srcw.at[wid, j], src_v)
      pltpu.sync_copy(dstw.at[wid, j], dst_v)
      pltpu.async_copy(hn.at[src_v], rows_v, sem).wait()
      pltpu.sync_copy(rows_v, agg_sh.at[dst_v], add=True)

    plsc.subcore_barrier()

    pltpu.sync_copy(agg_sh.at[pl.ds(r0, RPT)], parts.at[cid, pl.ds(r0, RPT)])

  params = pltpu.CompilerParams(use_tc_tiling_on_sc=False)
  return pl.kernel(body, out_type=out_type, mesh=mesh, scratch_types=scratch,
                   compiler_params=params)


DE = 144
_sc_agg_ext = _make_sc_agg(DE)
_sc_agg = _make_sc_agg(D)


RB = 1280
GRID = NPAD // RB


def _pre_body(x_ref, w_ref, o_ref):
  mm = jnp.dot(x_ref[...], w_ref[...], preferred_element_type=jnp.float32)
  o_ref[:, :D] = mm
  col = lax.broadcasted_iota(jnp.int32, (RB, DE - D), 1)
  o_ref[:, D:] = jnp.where(col == 0, 1.0, 0.0).astype(jnp.float32)


_tc_pre = pl.pallas_call(
    _pre_body,
    grid=(GRID,),
    in_specs=[pl.BlockSpec((RB, D), lambda i: (i, 0)),
              pl.BlockSpec((D, D), lambda i: (0, 0))],
    out_specs=pl.BlockSpec((RB, DE), lambda i: (i, 0)),
    out_shape=jax.ShapeDtypeStruct((NPAD, DE), jnp.float32),
)


_row = pl.BlockSpec((RB, D), lambda i: (i, 0))
_prow = pl.BlockSpec((NC, RB, D), lambda i: (0, i, 0))
_perow = pl.BlockSpec((NC, RB, DE), lambda i: (0, i, 0))
_irow = pl.BlockSpec((RB, 16), lambda i: (i, 0))
_wfull = pl.BlockSpec((D, D), lambda i: (0, 0))
_bfull = pl.BlockSpec((1, D), lambda i: (0, 0))


def _tcb1_body(x_ref, p_ref, ws, b, wn, h_out, hn_out, inv_out):
  p = p_ref[...]
  deg = jnp.maximum(p[0, :, D:D + 16] + p[1, :, D:D + 16], 1.0)
  inv = 1.0 / deg
  inv_out[...] = inv
  mean = (p[0, :, :D] + p[1, :, :D]) * inv[:, 0:1]
  h = jnp.dot(x_ref[...], ws[...], preferred_element_type=jnp.float32)
  h = jnp.maximum(h + mean + b[...], 0.0)
  h_out[...] = h
  hn_out[...] = jnp.dot(h, wn[...], preferred_element_type=jnp.float32)


_tcb1 = pl.pallas_call(
    _tcb1_body,
    grid=(GRID,),
    in_specs=[_row, _perow, _wfull, _bfull, _wfull],
    out_specs=[_row, _row, _irow],
    out_shape=[jax.ShapeDtypeStruct((NPAD, D), jnp.float32),
               jax.ShapeDtypeStruct((NPAD, D), jnp.float32),
               jax.ShapeDtypeStruct((NPAD, 16), jnp.float32)],
)


def _tcb2_body(x_ref, p_ref, inv_ref, ws, b, wn, h_out, hn_out):
  inv = inv_ref[...][:, 0:1]
  p = p_ref[...]
  mean = (p[0] + p[1]) * inv
  h = jnp.dot(x_ref[...], ws[...], preferred_element_type=jnp.float32)
  h = jnp.maximum(h + mean + b[...], 0.0)
  h_out[...] = h
  hn_out[...] = jnp.dot(h, wn[...], preferred_element_type=jnp.float32)


_tcb2 = pl.pallas_call(
    _tcb2_body,
    grid=(GRID,),
    in_specs=[_row, _prow, _irow, _wfull, _bfull, _wfull],
    out_specs=[_row, _row],
    out_shape=[jax.ShapeDtypeStruct((NPAD, D), jnp.float32),
               jax.ShapeDtypeStruct((NPAD, D), jnp.float32)],
)


def _tcc_body(x_ref, p_ref, inv_ref, ws, b, o_ref):
  inv = inv_ref[...][:, 0:1]
  p = p_ref[...]
  mean = (p[0] + p[1]) * inv
  h = jnp.dot(x_ref[...], ws[...], preferred_element_type=jnp.float32)
  o_ref[...] = h + mean + b[...]


_tcc = pl.pallas_call(
    _tcc_body,
    grid=(GRID,),
    in_specs=[_row, _prow, _irow, _wfull, _bfull],
    out_specs=_row,
    out_shape=jax.ShapeDtypeStruct((NPAD, D), jnp.float32),
)


def kernel(x, edge_index, W_self1, W_neigh1, b1, W_self2, W_neigh2, b2,
           W_self3, W_neigh3, b3):
  src = edge_index[0].astype(jnp.int32)
  dst = edge_index[1].astype(jnp.int32)
  pad = EPAD - E
  srcw = jnp.concatenate([src, jnp.zeros((pad,), jnp.int32)]).reshape(NW, CH, CHUNK)
  dstw = jnp.concatenate([dst, jnp.full((pad,), DUMMY, jnp.int32)]).reshape(NW, CH, CHUNK)
  xp = jnp.zeros((NPAD, D), jnp.float32).at[:N].set(x)
  wn3p = jnp.zeros((D, D), jnp.float32).at[:, :OUT].set(W_neigh3)
  ws3p = jnp.zeros((D, D), jnp.float32).at[:, :OUT].set(W_self3)
  b1r = b1.reshape(1, D)
  b2r = b2.reshape(1, D)
  b3p = jnp.zeros((1, D), jnp.float32).at[0, :OUT].set(b3)
  hn1 = _tc_pre(xp, W_neigh1)
  parts1, = _sc_agg_ext(hn1, srcw, dstw)
  h1, hn2, inv = _tcb1(xp, parts1, W_self1, b1r, W_neigh2)
  parts2, = _sc_agg(hn2, srcw, dstw)
  h2, hn3 = _tcb2(h1, parts2, inv, W_self2, b2r, wn3p)
  parts3, = _sc_agg(hn3, srcw, dstw)
  outp = _tcc(h2, parts3, inv, ws3p, b3p)
  return outp[:N, :OUT]


# speedup vs baseline: 1.4369x; 1.0011x over previous
import jax
import jax.numpy as jnp
from jax import lax
from jax.experimental import pallas as pl
from jax.experimental.pallas import tpu as pltpu
from jax.experimental.pallas import tpu_sc as plsc

N = 10000
E = 320000
D = 128
OUT = 47
NPAD = 10240
NC, NS = 2, 16
NW = NC * NS
CHUNK = 128
CH = -(-E // (NW * CHUNK))
EPAD = NW * CH * CHUNK
DUMMY = N
RPT = NPAD // NS
ZR = 128


def _make_sc_agg(width):
  mesh = plsc.VectorSubcoreMesh(core_axis_name="c", subcore_axis_name="s")
  out_type = [jax.ShapeDtypeStruct((NC, NPAD, width), jnp.float32)]
  scratch = [
      pltpu.VMEM((CHUNK,), jnp.int32),
      pltpu.VMEM((CHUNK,), jnp.int32),
      pltpu.VMEM((CHUNK, width), jnp.float32),
      pltpu.VMEM((ZR, width), jnp.float32),
      pltpu.VMEM_SHARED((NPAD, width), jnp.float32),
      pltpu.SemaphoreType.DMA,
  ]

  def body(hn, srcw, dstw, parts, src_v, dst_v, rows_v, zbuf, agg_sh, sem):
    cid = lax.axis_index("c")
    sid = lax.axis_index("s")
    wid = cid * NS + sid
    zero16 = jnp.zeros((16,), jnp.float32)

    @pl.loop(0, ZR)
    def _(i):
      for j in range(width // 16):
        zbuf[i, pl.ds(j * 16, 16)] = zero16

    r0 = sid * RPT

    @pl.loop(0, RPT // ZR)
    def _(k):
      pltpu.sync_copy(zbuf, agg_sh.at[pl.ds(r0 + k * ZR, ZR)])

    plsc.subcore_barrier()

    @pl.loop(0, CH)
    def _(j):
      pltpu.sync_copy(srcw.at[wid, j], src_v)
      pltpu.sync_copy(dstw.at[wid, j], dst_v)
      pltpu.async_copy(hn.at[src_v], rows_v, sem).wait()
      pltpu.sync_copy(rows_v, agg_sh.at[dst_v], add=True)

    plsc.subcore_barrier()

    pltpu.sync_copy(agg_sh.at[pl.ds(r0, RPT)], parts.at[cid, pl.ds(r0, RPT)])

  params = pltpu.CompilerParams(use_tc_tiling_on_sc=False)
  return pl.kernel(body, out_type=out_type, mesh=mesh, scratch_types=scratch,
                   compiler_params=params)


DE = 144
_sc_agg_ext = _make_sc_agg(DE)
_sc_agg = _make_sc_agg(D)


RB = 1280
GRID = NPAD // RB


def _pre_body(x_ref, w_ref, o_ref):
  mm = jnp.dot(x_ref[...], w_ref[...], preferred_element_type=jnp.float32)
  o_ref[:, :D] = mm
  col = lax.broadcasted_iota(jnp.int32, (RB, DE - D), 1)
  o_ref[:, D:] = jnp.where(col == 0, 1.0, 0.0).astype(jnp.float32)


_tc_pre = pl.pallas_call(
    _pre_body,
    grid=(GRID,),
    in_specs=[pl.BlockSpec((RB, D), lambda i: (i, 0)),
              pl.BlockSpec((D, D), lambda i: (0, 0))],
    out_specs=pl.BlockSpec((RB, DE), lambda i: (i, 0)),
    out_shape=jax.ShapeDtypeStruct((NPAD, DE), jnp.float32),
)


_row = pl.BlockSpec((RB, D), lambda i: (i, 0))
_prow = pl.BlockSpec((NC, RB, D), lambda i: (0, i, 0))
_perow = pl.BlockSpec((NC, RB, DE), lambda i: (0, i, 0))
_irow = pl.BlockSpec((RB, 16), lambda i: (i, 0))
_wfull = pl.BlockSpec((D, D), lambda i: (0, 0))
_bfull = pl.BlockSpec((1, D), lambda i: (0, 0))


def _tcb1_body(x_ref, p_ref, ws, b, wn, h_out, hn_out, inv_out):
  p = p_ref[...]
  deg = jnp.maximum(p[0, :, D:D + 16] + p[1, :, D:D + 16], 1.0)
  inv = 1.0 / deg
  inv_out[...] = inv
  mean = (p[0, :, :D] + p[1, :, :D]) * inv[:, 0:1]
  h = jnp.dot(x_ref[...], ws[...], preferred_element_type=jnp.float32)
  h = jnp.maximum(h + mean + b[...], 0.0)
  h_out[...] = h
  hn_out[...] = jnp.dot(h, wn[...], preferred_element_type=jnp.float32)


_tcb1 = pl.pallas_call(
    _tcb1_body,
    grid=(GRID,),
    in_specs=[_row, _perow, _wfull, _bfull, _wfull],
    out_specs=[_row, _row, _irow],
    out_shape=[jax.ShapeDtypeStruct((NPAD, D), jnp.float32),
               jax.ShapeDtypeStruct((NPAD, D), jnp.float32),
               jax.ShapeDtypeStruct((NPAD, 16), jnp.float32)],
)


def _tcb2_body(x_ref, p_ref, inv_ref, ws, b, wn, h_out, hn_out):
  inv = inv_ref[...][:, 0:1]
  p = p_ref[...]
  mean = (p[0] + p[1]) * inv
  h = jnp.dot(x_ref[...], ws[...], preferred_element_type=jnp.float32)
  h = jnp.maximum(h + mean + b[...], 0.0)
  h_out[...] = h
  hn_out[...] = jnp.dot(h, wn[...], preferred_element_type=jnp.float32)


_tcb2 = pl.pallas_call(
    _tcb2_body,
    grid=(GRID,),
    in_specs=[_row, _prow, _irow, _wfull, _bfull, _wfull],
    out_specs=[_row, _row],
    out_shape=[jax.ShapeDtypeStruct((NPAD, D), jnp.float32),
               jax.ShapeDtypeStruct((NPAD, D), jnp.float32)],
)


def _tcc_body(x_ref, p_ref, inv_ref, ws, b, o_ref):
  inv = inv_ref[...][:, 0:1]
  p = p_ref[...]
  mean = (p[0] + p[1]) * inv
  h = jnp.dot(x_ref[...], ws[...], preferred_element_type=jnp.float32)
  o_ref[...] = h + mean + b[...]


_tcc = pl.pallas_call(
    _tcc_body,
    grid=(GRID,),
    in_specs=[_row, _prow, _irow, _wfull, _bfull],
    out_specs=_row,
    out_shape=jax.ShapeDtypeStruct((NPAD, D), jnp.float32),
)


def kernel(x, edge_index, W_self1, W_neigh1, b1, W_self2, W_neigh2, b2,
           W_self3, W_neigh3, b3):
  src = edge_index[0].astype(jnp.int32)
  dst = edge_index[1].astype(jnp.int32)
  pad = EPAD - E
  srcw = jnp.concatenate([src, jnp.zeros((pad,), jnp.int32)]).reshape(NW, CH, CHUNK)
  dstw = jnp.concatenate([dst, jnp.full((pad,), DUMMY, jnp.int32)]).reshape(NW, CH, CHUNK)
  xp = jnp.zeros((NPAD, D), jnp.float32).at[:N].set(x)
  wn3p = jnp.zeros((D, D), jnp.float32).at[:, :OUT].set(W_neigh3)
  ws3p = jnp.zeros((D, D), jnp.float32).at[:, :OUT].set(W_self3)
  b1r = b1.reshape(1, D)
  b2r = b2.reshape(1, D)
  b3p = jnp.zeros((1, D), jnp.float32).at[0, :OUT].set(b3)
  hn1 = _tc_pre(xp, W_neigh1)
  parts1, = _sc_agg_ext(hn1, srcw, dstw)
  h1, hn2, inv = _tcb1(xp, parts1, W_self1, b1r, W_neigh2)
  parts2, = _sc_agg(hn2, srcw, dstw)
  h2, hn3 = _tcb2(h1, parts2, inv, W_self2, b2r, wn3p)
  parts3, = _sc_agg(hn3, srcw, dstw)
  outp = _tcc(h2, parts3, inv, ws3p, b3p)
  return outp[:N, :OUT]


# speedup vs baseline: 1.6798x; 1.1690x over previous
import jax
import jax.numpy as jnp
from jax import lax
from jax.experimental import pallas as pl
from jax.experimental.pallas import tpu as pltpu
from jax.experimental.pallas import tpu_sc as plsc

N = 10000
E = 320000
D = 128
OUT = 47
NPAD = 10240
NC, NS = 2, 16
NW = NC * NS
CHUNK = 128
CH = -(-E // (NW * CHUNK))
EPAD = NW * CH * CHUNK
DUMMY = N
RPT = NPAD // NS
ZR = 128


def _make_sc_agg(width, nbuf, zr):
  mesh = plsc.VectorSubcoreMesh(core_axis_name="c", subcore_axis_name="s")
  out_type = [jax.ShapeDtypeStruct((NC, NPAD, width), jnp.float32)]
  scratch = (
      [pltpu.VMEM((CHUNK,), jnp.int32)] * nbuf +
      [pltpu.VMEM((CHUNK,), jnp.int32)] * nbuf +
      [pltpu.VMEM((CHUNK, width), jnp.float32)] * nbuf +
      [pltpu.VMEM((zr, width), jnp.float32),
       pltpu.VMEM_SHARED((NPAD, width), jnp.float32)] +
      [pltpu.SemaphoreType.DMA] * nbuf
  )

  def body(hn, srcw, dstw, parts, *scr):
    src_v = scr[0:nbuf]
    dst_v = scr[nbuf:2 * nbuf]
    rows_v = scr[2 * nbuf:3 * nbuf]
    zbuf = scr[3 * nbuf]
    agg_sh = scr[3 * nbuf + 1]
    sem = scr[3 * nbuf + 2:3 * nbuf + 2 + nbuf]
    cid = lax.axis_index("c")
    sid = lax.axis_index("s")
    wid = cid * NS + sid
    zero16 = jnp.zeros((16,), jnp.float32)

    @pl.loop(0, zr)
    def _(i):
      for j in range(width // 16):
        zbuf[i, pl.ds(j * 16, 16)] = zero16

    r0 = sid * RPT

    @pl.loop(0, RPT // zr)
    def _(k):
      pltpu.sync_copy(zbuf, agg_sh.at[pl.ds(r0 + k * zr, zr)])

    plsc.subcore_barrier()

    if nbuf == 1:
      @pl.loop(0, CH)
      def _(j):
        pltpu.sync_copy(srcw.at[wid, j], src_v[0])
        pltpu.sync_copy(dstw.at[wid, j], dst_v[0])
        pltpu.async_copy(hn.at[src_v[0]], rows_v[0], sem[0]).wait()
        pltpu.sync_copy(rows_v[0], agg_sh.at[dst_v[0]], add=True)
    else:
      for b in range(2):
        pltpu.sync_copy(srcw.at[wid, b], src_v[b])
        pltpu.sync_copy(dstw.at[wid, b], dst_v[b])
        pltpu.async_copy(hn.at[src_v[b]], rows_v[b], sem[b])

      @pl.loop(0, (CH - 1) // 2 - 1)
      def _(j):
        for b in range(2):
          nxt = j * 2 + b + 2
          pltpu.make_async_copy(hn.at[src_v[b]], rows_v[b], sem[b]).wait()
          pltpu.sync_copy(rows_v[b], agg_sh.at[dst_v[b]], add=True)
          pltpu.sync_copy(srcw.at[wid, nxt], src_v[b])
          pltpu.sync_copy(dstw.at[wid, nxt], dst_v[b])
          pltpu.async_copy(hn.at[src_v[b]], rows_v[b], sem[b])

      for b in range(2):
        pltpu.make_async_copy(hn.at[src_v[b]], rows_v[b], sem[b]).wait()
        pltpu.sync_copy(rows_v[b], agg_sh.at[dst_v[b]], add=True)

      if CH % 2 == 1:
        pltpu.sync_copy(srcw.at[wid, CH - 1], src_v[0])
        pltpu.sync_copy(dstw.at[wid, CH - 1], dst_v[0])
        pltpu.async_copy(hn.at[src_v[0]], rows_v[0], sem[0]).wait()
        pltpu.sync_copy(rows_v[0], agg_sh.at[dst_v[0]], add=True)

    plsc.subcore_barrier()

    pltpu.sync_copy(agg_sh.at[pl.ds(r0, RPT)], parts.at[cid, pl.ds(r0, RPT)])

  params = pltpu.CompilerParams(use_tc_tiling_on_sc=False)
  return pl.kernel(body, out_type=out_type, mesh=mesh, scratch_types=scratch,
                   compiler_params=params)


DE = 144
_sc_agg_ext = _make_sc_agg(DE, 1, ZR)
_sc_agg = _make_sc_agg(D, 2, 32)


RB = 1280
GRID = NPAD // RB


def _pre_body(x_ref, w_ref, o_ref):
  mm = jnp.dot(x_ref[...], w_ref[...], preferred_element_type=jnp.float32)
  o_ref[:, :D] = mm
  col = lax.broadcasted_iota(jnp.int32, (RB, DE - D), 1)
  o_ref[:, D:] = jnp.where(col == 0, 1.0, 0.0).astype(jnp.float32)


_tc_pre = pl.pallas_call(
    _pre_body,
    grid=(GRID,),
    in_specs=[pl.BlockSpec((RB, D), lambda i: (i, 0)),
              pl.BlockSpec((D, D), lambda i: (0, 0))],
    out_specs=pl.BlockSpec((RB, DE), lambda i: (i, 0)),
    out_shape=jax.ShapeDtypeStruct((NPAD, DE), jnp.float32),
)


_row = pl.BlockSpec((RB, D), lambda i: (i, 0))
_prow = pl.BlockSpec((NC, RB, D), lambda i: (0, i, 0))
_perow = pl.BlockSpec((NC, RB, DE), lambda i: (0, i, 0))
_irow = pl.BlockSpec((RB, 16), lambda i: (i, 0))
_wfull = pl.BlockSpec((D, D), lambda i: (0, 0))
_bfull = pl.BlockSpec((1, D), lambda i: (0, 0))


def _tcb1_body(x_ref, p_ref, ws, b, wn, h_out, hn_out, inv_out):
  p = p_ref[...]
  deg = jnp.maximum(p[0, :, D:D + 16] + p[1, :, D:D + 16], 1.0)
  inv = 1.0 / deg
  inv_out[...] = inv
  mean = (p[0, :, :D] + p[1, :, :D]) * inv[:, 0:1]
  h = jnp.dot(x_ref[...], ws[...], preferred_element_type=jnp.float32)
  h = jnp.maximum(h + mean + b[...], 0.0)
  h_out[...] = h
  hn_out[...] = jnp.dot(h, wn[...], preferred_element_type=jnp.float32)


_tcb1 = pl.pallas_call(
    _tcb1_body,
    grid=(GRID,),
    in_specs=[_row, _perow, _wfull, _bfull, _wfull],
    out_specs=[_row, _row, _irow],
    out_shape=[jax.ShapeDtypeStruct((NPAD, D), jnp.float32),
               jax.ShapeDtypeStruct((NPAD, D), jnp.float32),
               jax.ShapeDtypeStruct((NPAD, 16), jnp.float32)],
)


def _tcb2_body(x_ref, p_ref, inv_ref, ws, b, wn, h_out, hn_out):
  inv = inv_ref[...][:, 0:1]
  p = p_ref[...]
  mean = (p[0] + p[1]) * inv
  h = jnp.dot(x_ref[...], ws[...], preferred_element_type=jnp.float32)
  h = jnp.maximum(h + mean + b[...], 0.0)
  h_out[...] = h
  hn_out[...] = jnp.dot(h, wn[...], preferred_element_type=jnp.float32)


_tcb2 = pl.pallas_call(
    _tcb2_body,
    grid=(GRID,),
    in_specs=[_row, _prow, _irow, _wfull, _bfull, _wfull],
    out_specs=[_row, _row],
    out_shape=[jax.ShapeDtypeStruct((NPAD, D), jnp.float32),
               jax.ShapeDtypeStruct((NPAD, D), jnp.float32)],
)


def _tcc_body(x_ref, p_ref, inv_ref, ws, b, o_ref):
  inv = inv_ref[...][:, 0:1]
  p = p_ref[...]
  mean = (p[0] + p[1]) * inv
  h = jnp.dot(x_ref[...], ws[...], preferred_element_type=jnp.float32)
  o_ref[...] = h + mean + b[...]


_tcc = pl.pallas_call(
    _tcc_body,
    grid=(GRID,),
    in_specs=[_row, _prow, _irow, _wfull, _bfull],
    out_specs=_row,
    out_shape=jax.ShapeDtypeStruct((NPAD, D), jnp.float32),
)


def kernel(x, edge_index, W_self1, W_neigh1, b1, W_self2, W_neigh2, b2,
           W_self3, W_neigh3, b3):
  src = edge_index[0].astype(jnp.int32)
  dst = edge_index[1].astype(jnp.int32)
  pad = EPAD - E
  srcw = jnp.concatenate([src, jnp.zeros((pad,), jnp.int32)]).reshape(NW, CH, CHUNK)
  dstw = jnp.concatenate([dst, jnp.full((pad,), DUMMY, jnp.int32)]).reshape(NW, CH, CHUNK)
  xp = jnp.zeros((NPAD, D), jnp.float32).at[:N].set(x)
  wn3p = jnp.zeros((D, D), jnp.float32).at[:, :OUT].set(W_neigh3)
  ws3p = jnp.zeros((D, D), jnp.float32).at[:, :OUT].set(W_self3)
  b1r = b1.reshape(1, D)
  b2r = b2.reshape(1, D)
  b3p = jnp.zeros((1, D), jnp.float32).at[0, :OUT].set(b3)
  hn1 = _tc_pre(xp, W_neigh1)
  parts1, = _sc_agg_ext(hn1, srcw, dstw)
  h1, hn2, inv = _tcb1(xp, parts1, W_self1, b1r, W_neigh2)
  parts2, = _sc_agg(hn2, srcw, dstw)
  h2, hn3 = _tcb2(h1, parts2, inv, W_self2, b2r, wn3p)
  parts3, = _sc_agg(hn3, srcw, dstw)
  outp = _tcc(h2, parts3, inv, ws3p, b3p)
  return outp[:N, :OUT]


# speedup vs baseline: 1.8455x; 1.0986x over previous
import jax
import jax.numpy as jnp
from jax import lax
from jax.experimental import pallas as pl
from jax.experimental.pallas import tpu as pltpu
from jax.experimental.pallas import tpu_sc as plsc

N = 10000
E = 320000
D = 128
OUT = 47
NPAD = 10240
NC, NS = 2, 16
NW = NC * NS
CHUNK = 128
CH = -(-E // (NW * CHUNK))
EPAD = NW * CH * CHUNK
DUMMY = N
RPT = NPAD // NS
ZR = 128


def _make_sc_agg(width, nbuf, zr):
  mesh = plsc.VectorSubcoreMesh(core_axis_name="c", subcore_axis_name="s")
  out_type = [jax.ShapeDtypeStruct((NC, NPAD, width), jnp.float32)]
  scratch = (
      [pltpu.VMEM((CHUNK,), jnp.int32)] * nbuf +
      [pltpu.VMEM((CHUNK,), jnp.int32)] * nbuf +
      [pltpu.VMEM((CHUNK, width), jnp.float32)] * nbuf +
      [pltpu.VMEM((zr, width), jnp.float32),
       pltpu.VMEM_SHARED((NPAD, width), jnp.float32)] +
      [pltpu.SemaphoreType.DMA] * nbuf
  )

  def body(hn, srcw, dstw, parts, *scr):
    src_v = scr[0:nbuf]
    dst_v = scr[nbuf:2 * nbuf]
    rows_v = scr[2 * nbuf:3 * nbuf]
    zbuf = scr[3 * nbuf]
    agg_sh = scr[3 * nbuf + 1]
    sem = scr[3 * nbuf + 2:3 * nbuf + 2 + nbuf]
    cid = lax.axis_index("c")
    sid = lax.axis_index("s")
    wid = cid * NS + sid
    zero16 = jnp.zeros((16,), jnp.float32)

    @pl.loop(0, zr)
    def _(i):
      for j in range(width // 16):
        zbuf[i, pl.ds(j * 16, 16)] = zero16

    r0 = sid * RPT

    @pl.loop(0, RPT // zr)
    def _(k):
      pltpu.sync_copy(zbuf, agg_sh.at[pl.ds(r0 + k * zr, zr)])

    plsc.subcore_barrier()

    if nbuf == 1:
      @pl.loop(0, CH)
      def _(j):
        pltpu.sync_copy(srcw.at[wid, j], src_v[0])
        pltpu.sync_copy(dstw.at[wid, j], dst_v[0])
        pltpu.async_copy(hn.at[src_v[0]], rows_v[0], sem[0]).wait()
        pltpu.sync_copy(rows_v[0], agg_sh.at[dst_v[0]], add=True)
    else:
      for b in range(2):
        pltpu.sync_copy(srcw.at[wid, b], src_v[b])
        pltpu.sync_copy(dstw.at[wid, b], dst_v[b])
        pltpu.async_copy(hn.at[src_v[b]], rows_v[b], sem[b])

      @pl.loop(0, (CH - 1) // 2 - 1)
      def _(j):
        for b in range(2):
          nxt = j * 2 + b + 2
          pltpu.make_async_copy(hn.at[src_v[b]], rows_v[b], sem[b]).wait()
          pltpu.sync_copy(rows_v[b], agg_sh.at[dst_v[b]], add=True)
          pltpu.sync_copy(srcw.at[wid, nxt], src_v[b])
          pltpu.sync_copy(dstw.at[wid, nxt], dst_v[b])
          pltpu.async_copy(hn.at[src_v[b]], rows_v[b], sem[b])

      for b in range(2):
        pltpu.make_async_copy(hn.at[src_v[b]], rows_v[b], sem[b]).wait()
        pltpu.sync_copy(rows_v[b], agg_sh.at[dst_v[b]], add=True)

      if CH % 2 == 1:
        pltpu.sync_copy(srcw.at[wid, CH - 1], src_v[0])
        pltpu.sync_copy(dstw.at[wid, CH - 1], dst_v[0])
        pltpu.async_copy(hn.at[src_v[0]], rows_v[0], sem[0]).wait()
        pltpu.sync_copy(rows_v[0], agg_sh.at[dst_v[0]], add=True)

    plsc.subcore_barrier()

    pltpu.sync_copy(agg_sh.at[pl.ds(r0, RPT)], parts.at[cid, pl.ds(r0, RPT)])

  params = pltpu.CompilerParams(use_tc_tiling_on_sc=False)
  return pl.kernel(body, out_type=out_type, mesh=mesh, scratch_types=scratch,
                   compiler_params=params)


DE = 144
_sc_agg_ext = _make_sc_agg(DE, 2, 8)
_sc_agg = _make_sc_agg(D, 2, 32)


RB = 1280
GRID = NPAD // RB


def _pre_body(x_ref, w_ref, o_ref):
  mm = jnp.dot(x_ref[...], w_ref[...], preferred_element_type=jnp.float32)
  o_ref[:, :D] = mm
  col = lax.broadcasted_iota(jnp.int32, (RB, DE - D), 1)
  o_ref[:, D:] = jnp.where(col == 0, 1.0, 0.0).astype(jnp.float32)


_tc_pre = pl.pallas_call(
    _pre_body,
    grid=(GRID,),
    in_specs=[pl.BlockSpec((RB, D), lambda i: (i, 0)),
              pl.BlockSpec((D, D), lambda i: (0, 0))],
    out_specs=pl.BlockSpec((RB, DE), lambda i: (i, 0)),
    out_shape=jax.ShapeDtypeStruct((NPAD, DE), jnp.float32),
)


_row = pl.BlockSpec((RB, D), lambda i: (i, 0))
_prow = pl.BlockSpec((NC, RB, D), lambda i: (0, i, 0))
_perow = pl.BlockSpec((NC, RB, DE), lambda i: (0, i, 0))
_irow = pl.BlockSpec((RB, 16), lambda i: (i, 0))
_wfull = pl.BlockSpec((D, D), lambda i: (0, 0))
_bfull = pl.BlockSpec((1, D), lambda i: (0, 0))


def _tcb1_body(x_ref, p_ref, ws, b, wn, h_out, hn_out, inv_out):
  p = p_ref[...]
  deg = jnp.maximum(p[0, :, D:D + 16] + p[1, :, D:D + 16], 1.0)
  inv = 1.0 / deg
  inv_out[...] = inv
  mean = (p[0, :, :D] + p[1, :, :D]) * inv[:, 0:1]
  h = jnp.dot(x_ref[...], ws[...], preferred_element_type=jnp.float32)
  h = jnp.maximum(h + mean + b[...], 0.0)
  h_out[...] = h
  hn_out[...] = jnp.dot(h, wn[...], preferred_element_type=jnp.float32)


_tcb1 = pl.pallas_call(
    _tcb1_body,
    grid=(GRID,),
    in_specs=[_row, _perow, _wfull, _bfull, _wfull],
    out_specs=[_row, _row, _irow],
    out_shape=[jax.ShapeDtypeStruct((NPAD, D), jnp.float32),
               jax.ShapeDtypeStruct((NPAD, D), jnp.float32),
               jax.ShapeDtypeStruct((NPAD, 16), jnp.float32)],
)


def _tcb2_body(x_ref, p_ref, inv_ref, ws, b, wn, h_out, hn_out):
  inv = inv_ref[...][:, 0:1]
  p = p_ref[...]
  mean = (p[0] + p[1]) * inv
  h = jnp.dot(x_ref[...], ws[...], preferred_element_type=jnp.float32)
  h = jnp.maximum(h + mean + b[...], 0.0)
  h_out[...] = h
  hn_out[...] = jnp.dot(h, wn[...], preferred_element_type=jnp.float32)


_tcb2 = pl.pallas_call(
    _tcb2_body,
    grid=(GRID,),
    in_specs=[_row, _prow, _irow, _wfull, _bfull, _wfull],
    out_specs=[_row, _row],
    out_shape=[jax.ShapeDtypeStruct((NPAD, D), jnp.float32),
               jax.ShapeDtypeStruct((NPAD, D), jnp.float32)],
)


def _tcc_body(x_ref, p_ref, inv_ref, ws, b, o_ref):
  inv = inv_ref[...][:, 0:1]
  p = p_ref[...]
  mean = (p[0] + p[1]) * inv
  h = jnp.dot(x_ref[...], ws[...], preferred_element_type=jnp.float32)
  o_ref[...] = h + mean + b[...]


_tcc = pl.pallas_call(
    _tcc_body,
    grid=(GRID,),
    in_specs=[_row, _prow, _irow, _wfull, _bfull],
    out_specs=_row,
    out_shape=jax.ShapeDtypeStruct((NPAD, D), jnp.float32),
)


def kernel(x, edge_index, W_self1, W_neigh1, b1, W_self2, W_neigh2, b2,
           W_self3, W_neigh3, b3):
  src = edge_index[0].astype(jnp.int32)
  dst = edge_index[1].astype(jnp.int32)
  pad = EPAD - E
  srcw = jnp.concatenate([src, jnp.zeros((pad,), jnp.int32)]).reshape(NW, CH, CHUNK)
  dstw = jnp.concatenate([dst, jnp.full((pad,), DUMMY, jnp.int32)]).reshape(NW, CH, CHUNK)
  xp = jnp.zeros((NPAD, D), jnp.float32).at[:N].set(x)
  wn3p = jnp.zeros((D, D), jnp.float32).at[:, :OUT].set(W_neigh3)
  ws3p = jnp.zeros((D, D), jnp.float32).at[:, :OUT].set(W_self3)
  b1r = b1.reshape(1, D)
  b2r = b2.reshape(1, D)
  b3p = jnp.zeros((1, D), jnp.float32).at[0, :OUT].set(b3)
  hn1 = _tc_pre(xp, W_neigh1)
  parts1, = _sc_agg_ext(hn1, srcw, dstw)
  h1, hn2, inv = _tcb1(xp, parts1, W_self1, b1r, W_neigh2)
  parts2, = _sc_agg(hn2, srcw, dstw)
  h2, hn3 = _tcb2(h1, parts2, inv, W_self2, b2r, wn3p)
  parts3, = _sc_agg(hn3, srcw, dstw)
  outp = _tcc(h2, parts3, inv, ws3p, b3p)
  return outp[:N, :OUT]
